# baseline (speedup 1.0000x reference)
import jax
import jax.numpy as jnp
from jax.experimental import pallas as pl

L_MAX = 2
NO = (L_MAX + 1) ** 2
F = NO * 16
TN = 512


def _so3_body(x_ref, sh_ref, k_ref, b_ref, out_ref):
    x = x_ref[...].astype(jnp.bfloat16)
    sh = sh_ref[...]
    acc = jnp.zeros((x.shape[0], F), dtype=jnp.float32)
    for me in range(NO):
        s = jnp.dot(sh, b_ref[me], preferred_element_type=jnp.float32)
        y = jnp.dot(x, k_ref[me], preferred_element_type=jnp.float32)
        acc = acc + y * s
    out_ref[...] = acc


def kernel(x, sh, weight, CG_vals, M1, M2, seg1_ids, l_ind, seg2_ids):
    n = x.shape[0]
    w_e = weight[0][l_ind[seg1_ids]]
    mo_e = seg2_ids[seg1_ids]
    k = jnp.zeros((NO, NO, 16, NO, 16), dtype=jnp.float32)
    k = k.at[M2, M1, :, mo_e, :].add(CG_vals[:, None, None] * w_e)
    k = k.reshape(NO, F, F).astype(jnp.bfloat16)

    b = jnp.zeros((NO, NO, F), dtype=jnp.float32)
    b = b.at[jnp.arange(NO), jnp.arange(NO), :].set(1.0)

    x_flat = x.reshape(n, F)
    out = pl.pallas_call(
        _so3_body,
        grid=(n // TN,),
        in_specs=[
            pl.BlockSpec((TN, F), lambda i: (i, 0)),
            pl.BlockSpec((TN, NO), lambda i: (i, 0)),
            pl.BlockSpec((NO, F, F), lambda i: (0, 0, 0)),
            pl.BlockSpec((NO, NO, F), lambda i: (0, 0, 0)),
        ],
        out_specs=pl.BlockSpec((TN, F), lambda i: (i, 0)),
        out_shape=jax.ShapeDtypeStruct((n, F), jnp.float32),
    )(x_flat, sh, k, b)
    return out.reshape(n, NO, 16)


# speedup vs baseline: 1.0367x; 1.0367x over previous
import jax
import jax.numpy as jnp
from jax.experimental import pallas as pl

L_MAX = 2
NO = (L_MAX + 1) ** 2
F = NO * 16
TN = 1024


def _so3_body(x_ref, sh_ref, k_ref, b_ref, out_ref):
    x = x_ref[...].astype(jnp.bfloat16)
    sh = sh_ref[...]
    acc = jnp.zeros((x.shape[0], F), dtype=jnp.float32)
    for me in range(NO):
        s = jnp.dot(sh, b_ref[me], preferred_element_type=jnp.float32)
        y = jnp.dot(x, k_ref[me], preferred_element_type=jnp.float32)
        acc = acc + y * s
    out_ref[...] = acc


def kernel(x, sh, weight, CG_vals, M1, M2, seg1_ids, l_ind, seg2_ids):
    n = x.shape[0]
    w_e = weight[0][l_ind[seg1_ids]]
    mo_e = seg2_ids[seg1_ids]
    k = jnp.zeros((NO, NO, 16, NO, 16), dtype=jnp.float32)
    k = k.at[M2, M1, :, mo_e, :].add(CG_vals[:, None, None] * w_e)
    k = k.reshape(NO, F, F).astype(jnp.bfloat16)

    b = jnp.zeros((NO, NO, F), dtype=jnp.float32)
    b = b.at[jnp.arange(NO), jnp.arange(NO), :].set(1.0)

    x_flat = x.reshape(n, F)
    out = pl.pallas_call(
        _so3_body,
        grid=(n // TN,),
        in_specs=[
            pl.BlockSpec((TN, F), lambda i: (i, 0)),
            pl.BlockSpec((TN, NO), lambda i: (i, 0)),
            pl.BlockSpec((NO, F, F), lambda i: (0, 0, 0)),
            pl.BlockSpec((NO, NO, F), lambda i: (0, 0, 0)),
        ],
        out_specs=pl.BlockSpec((TN, F), lambda i: (i, 0)),
        out_shape=jax.ShapeDtypeStruct((n, F), jnp.float32),
    )(x_flat, sh, k, b)
    return out.reshape(n, NO, 16)


# speedup vs baseline: 1.1553x; 1.1145x over previous
import jax
import jax.numpy as jnp
from jax import lax
from jax.experimental import pallas as pl
from jax.experimental.pallas import tpu as pltpu

L_MAX = 2
NO = (L_MAX + 1) ** 2
F = NO * 16
TN = 2048
NBUF = 4


def _so3_body(x_hbm, sh_hbm, k_ref, o_hbm, xbuf, shbuf, obuf,
              x_sems, sh_sems, o_sems):
    nchunks = x_hbm.shape[0] // TN

    def x_copy(i, slot):
        return pltpu.make_async_copy(
            x_hbm.at[pl.ds(i * TN, TN)], xbuf.at[slot], x_sems.at[slot])

    def sh_copy(i, slot):
        return pltpu.make_async_copy(
            sh_hbm.at[pl.ds(i * TN, TN)], shbuf.at[slot], sh_sems.at[slot])

    def o_copy(i, slot):
        return pltpu.make_async_copy(
            obuf.at[slot], o_hbm.at[pl.ds(i * TN, TN)], o_sems.at[slot])

    for b in range(NBUF):
        x_copy(b, b).start()
        sh_copy(b, b).start()

    def step(i, _):
        slot = lax.rem(i, NBUF)
        x_copy(i, slot).wait()
        sh_copy(i, slot).wait()

        @pl.when(i >= NBUF)
        def _():
            o_copy(i - NBUF, slot).wait()

        x = xbuf[slot]
        sh = shbuf[slot]
        acc = jnp.zeros((TN, F), dtype=jnp.float32)
        for me in range(NO):
            xs = (x * sh[:, me:me + 1]).astype(jnp.bfloat16)
            acc = acc + jnp.dot(xs, k_ref[me],
                                preferred_element_type=jnp.float32)
        obuf[slot] = acc
        o_copy(i, slot).start()

        @pl.when(i + NBUF < nchunks)
        def _():
            x_copy(i + NBUF, slot).start()
            sh_copy(i + NBUF, slot).start()
        return 0

    lax.fori_loop(0, nchunks, step, 0)
    for b in range(NBUF):
        slot = lax.rem(nchunks - NBUF + b, NBUF)
        o_copy(nchunks - NBUF + b, slot).wait()


def kernel(x, sh, weight, CG_vals, M1, M2, seg1_ids, l_ind, seg2_ids):
    n = x.shape[0]
    w_e = weight[0][l_ind[seg1_ids]]
    mo_e = seg2_ids[seg1_ids]
    k = jnp.zeros((NO, NO, 16, NO, 16), dtype=jnp.float32)
    k = k.at[M2, M1, :, mo_e, :].add(CG_vals[:, None, None] * w_e)
    k = k.reshape(NO, F, F).astype(jnp.bfloat16)

    x_flat = x.reshape(n, F)
    out = pl.pallas_call(
        _so3_body,
        in_specs=[
            pl.BlockSpec(memory_space=pl.ANY),
            pl.BlockSpec(memory_space=pl.ANY),
            pl.BlockSpec(memory_space=pltpu.VMEM),
        ],
        out_specs=pl.BlockSpec(memory_space=pl.ANY),
        out_shape=jax.ShapeDtypeStruct((n, F), jnp.float32),
        scratch_shapes=[
            pltpu.VMEM((NBUF, TN, F), jnp.float32),
            pltpu.VMEM((NBUF, TN, NO), jnp.float32),
            pltpu.VMEM((NBUF, TN, F), jnp.float32),
            pltpu.SemaphoreType.DMA((NBUF,)),
            pltpu.SemaphoreType.DMA((NBUF,)),
            pltpu.SemaphoreType.DMA((NBUF,)),
        ],
    )(x_flat, sh, k)
    return out.reshape(n, NO, 16)


# speedup vs baseline: 1.2886x; 1.1153x over previous
import jax
import jax.numpy as jnp
from jax import lax
from jax.experimental import pallas as pl
from jax.experimental.pallas import tpu as pltpu

L_MAX = 2
NO = (L_MAX + 1) ** 2
F = NO * 16
TN = 2048
NBUF = 4


FP = 256


def _so3_body(x_hbm, sh_hbm, k_ref, o_hbm, xbuf, shbuf, obuf, xs_all,
              x_sems, sh_sems, o_sems):
    nchunks = x_hbm.shape[0] // TN

    def x_copy(i, slot):
        return pltpu.make_async_copy(
            x_hbm.at[pl.ds(i * TN, TN)], xbuf.at[slot], x_sems.at[slot])

    def sh_copy(i, slot):
        return pltpu.make_async_copy(
            sh_hbm.at[pl.ds(i * TN, TN)], shbuf.at[slot], sh_sems.at[slot])

    def o_copy(i, slot):
        return pltpu.make_async_copy(
            obuf.at[slot], o_hbm.at[pl.ds(i * TN, TN)], o_sems.at[slot])

    xs_all[...] = jnp.zeros((TN, NO * FP), dtype=jnp.bfloat16)

    for b in range(NBUF):
        x_copy(b, b).start()
        sh_copy(b, b).start()

    def step(i, _):
        slot = lax.rem(i, NBUF)
        x_copy(i, slot).wait()
        sh_copy(i, slot).wait()

        @pl.when(i >= NBUF)
        def _():
            o_copy(i - NBUF, slot).wait()

        x = xbuf[slot].astype(jnp.bfloat16)
        sh = shbuf[slot].astype(jnp.bfloat16)
        for me in range(NO):
            xs_all[:, me * FP:me * FP + F] = x * sh[:, me:me + 1]
        obuf[slot] = jnp.dot(xs_all[...], k_ref[...],
                             preferred_element_type=jnp.float32)
        o_copy(i, slot).start()

        @pl.when(i + NBUF < nchunks)
        def _():
            x_copy(i + NBUF, slot).start()
            sh_copy(i + NBUF, slot).start()
        return 0

    lax.fori_loop(0, nchunks, step, 0)
    for b in range(NBUF):
        slot = lax.rem(nchunks - NBUF + b, NBUF)
        o_copy(nchunks - NBUF + b, slot).wait()


def kernel(x, sh, weight, CG_vals, M1, M2, seg1_ids, l_ind, seg2_ids):
    n = x.shape[0]
    w_e = weight[0][l_ind[seg1_ids]]
    mo_e = seg2_ids[seg1_ids]
    k = jnp.zeros((NO, NO, 16, NO, 16), dtype=jnp.float32)
    k = k.at[M2, M1, :, mo_e, :].add(CG_vals[:, None, None] * w_e)
    kp = jnp.zeros((NO, FP, F), dtype=jnp.float32)
    kp = kp.at[:, :F, :].set(k.reshape(NO, F, F))
    kp = kp.reshape(NO * FP, F).astype(jnp.bfloat16)

    x_flat = x.reshape(n, F)
    out = pl.pallas_call(
        _so3_body,
        in_specs=[
            pl.BlockSpec(memory_space=pl.ANY),
            pl.BlockSpec(memory_space=pl.ANY),
            pl.BlockSpec(memory_space=pltpu.VMEM),
        ],
        out_specs=pl.BlockSpec(memory_space=pl.ANY),
        out_shape=jax.ShapeDtypeStruct((n, F), jnp.float32),
        scratch_shapes=[
            pltpu.VMEM((NBUF, TN, F), jnp.float32),
            pltpu.VMEM((NBUF, TN, NO), jnp.float32),
            pltpu.VMEM((NBUF, TN, F), jnp.float32),
            pltpu.VMEM((TN, NO * FP), jnp.bfloat16),
            pltpu.SemaphoreType.DMA((NBUF,)),
            pltpu.SemaphoreType.DMA((NBUF,)),
            pltpu.SemaphoreType.DMA((NBUF,)),
        ],
    )(x_flat, sh, kp)
    return out.reshape(n, NO, 16)
